# Initial kernel scaffold; baseline (speedup 1.0000x reference)
import jax
import jax.numpy as jnp
from jax.experimental import pallas as pl


def kernel(x, edge_index, W1, att_src1, att_dst1, b1, W2, att_src2, att_dst2, b2):
    raise NotImplementedError("write your pallas kernel here")


# speedup vs baseline: 33.1950x; 33.1950x over previous
import functools

import jax
import jax.numpy as jnp
import numpy as np
from jax import lax
from jax.experimental import pallas as pl
from jax.experimental.pallas import tpu as pltpu, tpu_sc as plsc

N = 10000
E = 320000
IN_CH = 128
HID = 8
HEADS = 8
OUT_CH = 16

NC, NS = 2, 16
NW = NC * NS
N_PAD = 10240
STRIPE = N_PAD // NS
EPT = 10240
E_PAD = EPT * NW
CH = 128
NCHUNK = EPT // CH
ROWB = 256
NBLK = N_PAD // ROWB

_sc_params = pltpu.CompilerParams(use_tc_tiling_on_sc=False)


@functools.lru_cache(maxsize=None)
def _sc_mesh():
    return plsc.VectorSubcoreMesh(core_axis_name="c", subcore_axis_name="s")


def _edge_scores_body(asrc_hbm, adst_hbm, src_hbm, dst_hbm, z16_hbm,
                      ex_hbm, den_hbm,
                      src_v, dst_v, as_v, ad_v, ex_v, den_sh, sem):
    cid = lax.axis_index("c")
    sid = lax.axis_index("s")
    wid = sid * NC + cid

    pltpu.sync_copy(src_hbm.at[wid], src_v)
    pltpu.sync_copy(dst_hbm.at[wid], dst_v)
    pltpu.sync_copy(z16_hbm, den_sh.at[pl.ds(sid * STRIPE, STRIPE)])
    plsc.subcore_barrier()

    for j in range(NCHUNK):
        pltpu.async_copy(asrc_hbm.at[src_v.at[j]], as_v, sem).wait()
        pltpu.async_copy(adst_hbm.at[dst_v.at[j]], ad_v, sem).wait()

        def eloop(i, _):
            s = as_v[i, :] + ad_v[i, :]
            ex_v[i, :] = jnp.exp(jnp.maximum(s, 0.2 * s))
            return ()
        lax.fori_loop(0, CH, eloop, ())

        pltpu.sync_copy(ex_v, den_sh.at[dst_v.at[j]], add=True)
        pltpu.sync_copy(ex_v, ex_hbm.at[wid, j])

    plsc.subcore_barrier()
    pltpu.sync_copy(den_sh.at[pl.ds(sid * STRIPE, STRIPE)],
                    den_hbm.at[cid, pl.ds(sid * STRIPE, STRIPE)])


@functools.lru_cache(maxsize=None)
def _edge_scores_kernel():
    return pl.kernel(
        _edge_scores_body,
        out_type=[
            jax.ShapeDtypeStruct((NW, NCHUNK, CH, 16), jnp.float32),
            jax.ShapeDtypeStruct((NC, N_PAD, 16), jnp.float32),
        ],
        mesh=_sc_mesh(),
        scratch_types=[
            pltpu.VMEM((NCHUNK, CH), jnp.int32),
            pltpu.VMEM((NCHUNK, CH), jnp.int32),
            pltpu.VMEM((CH, 16), jnp.float32),
            pltpu.VMEM((CH, 16), jnp.float32),
            pltpu.VMEM((CH, 16), jnp.float32),
            pltpu.VMEM_SHARED((N_PAD, 16), jnp.float32),
            pltpu.SemaphoreType.DMA,
        ],
        compiler_params=_sc_params,
    )


def _make_aggregate(D):
    DV = D // 16

    def body(h_hbm, den_hbm, ex_hbm, src_hbm, dst_hbm, zD_hbm,
             acc_out_hbm,
             src_v, dst_v, ex_v, den_v, rows_v, acc_sh, sem):
        cid = lax.axis_index("c")
        sid = lax.axis_index("s")
        wid = sid * NC + cid

        pltpu.sync_copy(src_hbm.at[wid], src_v)
        pltpu.sync_copy(dst_hbm.at[wid], dst_v)
        pltpu.sync_copy(zD_hbm, acc_sh.at[pl.ds(sid * STRIPE, STRIPE)])
        plsc.subcore_barrier()

        for j in range(NCHUNK):
            pltpu.sync_copy(ex_hbm.at[wid, j], ex_v)
            pltpu.async_copy(den_hbm.at[dst_v.at[j]], den_v, sem).wait()
            pltpu.async_copy(h_hbm.at[src_v.at[j]], rows_v, sem).wait()

            def mloop(i, _):
                c = ex_v[i, :] / (den_v[i, :] + 1e-16)
                for q in range(DV):
                    rows_v[i, pl.ds(q * 16, 16)] = rows_v[i, pl.ds(q * 16, 16)] * c
                return ()
            lax.fori_loop(0, CH, mloop, ())

            pltpu.sync_copy(rows_v, acc_sh.at[dst_v.at[j]], add=True)

        plsc.subcore_barrier()
        pltpu.sync_copy(acc_sh.at[pl.ds(sid * STRIPE, STRIPE)],
                        acc_out_hbm.at[cid, pl.ds(sid * STRIPE, STRIPE)])

    return pl.kernel(
        body,
        out_type=[jax.ShapeDtypeStruct((NC, N_PAD, D), jnp.float32)],
        mesh=_sc_mesh(),
        scratch_types=[
            pltpu.VMEM((NCHUNK, CH), jnp.int32),
            pltpu.VMEM((NCHUNK, CH), jnp.int32),
            pltpu.VMEM((CH, 16), jnp.float32),
            pltpu.VMEM((CH, 16), jnp.float32),
            pltpu.VMEM((CH, D), jnp.float32),
            pltpu.VMEM_SHARED((N_PAD, D), jnp.float32),
            pltpu.SemaphoreType.DMA,
        ],
        compiler_params=_sc_params,
    )


_aggregate = functools.lru_cache(maxsize=None)(_make_aggregate)


def _tc1_body(x_ref, w1_ref, asm_ref, adm_ref,
              h1_ref, as16_ref, ad16_ref, exs_ref):
    h1 = jnp.dot(x_ref[...], w1_ref[...], preferred_element_type=jnp.float32)
    h1_ref[...] = h1
    a_s = jnp.dot(h1, asm_ref[...], preferred_element_type=jnp.float32)
    a_d = jnp.dot(h1, adm_ref[...], preferred_element_type=jnp.float32)
    as16_ref[...] = a_s
    ad16_ref[...] = a_d
    s = a_s + a_d
    exs_ref[...] = jnp.exp(jnp.maximum(s, 0.2 * s))


def _tc1(x_pad, w1p, asm, adm):
    return pl.pallas_call(
        _tc1_body,
        grid=(NBLK,),
        in_specs=[
            pl.BlockSpec((ROWB, IN_CH), lambda i: (i, 0)),
            pl.BlockSpec((IN_CH, 64), lambda i: (0, 0)),
            pl.BlockSpec((64, 16), lambda i: (0, 0)),
            pl.BlockSpec((64, 16), lambda i: (0, 0)),
        ],
        out_specs=[
            pl.BlockSpec((ROWB, 64), lambda i: (i, 0)),
            pl.BlockSpec((ROWB, 16), lambda i: (i, 0)),
            pl.BlockSpec((ROWB, 16), lambda i: (i, 0)),
            pl.BlockSpec((ROWB, 16), lambda i: (i, 0)),
        ],
        out_shape=[
            jax.ShapeDtypeStruct((N_PAD, 64), jnp.float32),
            jax.ShapeDtypeStruct((N_PAD, 16), jnp.float32),
            jax.ShapeDtypeStruct((N_PAD, 16), jnp.float32),
            jax.ShapeDtypeStruct((N_PAD, 16), jnp.float32),
        ],
    )(x_pad, w1p, asm, adm)


def _tc2_body(acc_ref, h1_ref, exs_ref, den_ref, b1_ref, w2_ref,
              a2sm_ref, a2dm_ref,
              h2_ref, a2s_ref, a2d_ref, exs2_ref):
    coef = exs_ref[...] / (den_ref[...] + 1e-16)
    coef64 = jnp.concatenate([coef, coef, coef, coef], axis=1)
    out1 = acc_ref[...] + h1_ref[...] * coef64 + b1_ref[...]
    hin2 = jnp.where(out1 > 0, out1, jnp.exp(jnp.minimum(out1, 0.0)) - 1.0)
    h2 = jnp.dot(hin2, w2_ref[...], preferred_element_type=jnp.float32)
    h2_ref[...] = h2
    a2s = jnp.dot(h2, a2sm_ref[...], preferred_element_type=jnp.float32)
    a2d = jnp.dot(h2, a2dm_ref[...], preferred_element_type=jnp.float32)
    a2s_ref[...] = a2s
    a2d_ref[...] = a2d
    s = a2s + a2d
    exs2_ref[...] = jnp.exp(jnp.maximum(s, 0.2 * s))


def _tc2(acc1, h1p, exs1, den1, b1p, w2p, a2sm, a2dm):
    return pl.pallas_call(
        _tc2_body,
        grid=(NBLK,),
        in_specs=[
            pl.BlockSpec((ROWB, 64), lambda i: (i, 0)),
            pl.BlockSpec((ROWB, 64), lambda i: (i, 0)),
            pl.BlockSpec((ROWB, 16), lambda i: (i, 0)),
            pl.BlockSpec((ROWB, 16), lambda i: (i, 0)),
            pl.BlockSpec((1, 64), lambda i: (0, 0)),
            pl.BlockSpec((64, 16), lambda i: (0, 0)),
            pl.BlockSpec((16, 16), lambda i: (0, 0)),
            pl.BlockSpec((16, 16), lambda i: (0, 0)),
        ],
        out_specs=[
            pl.BlockSpec((ROWB, 16), lambda i: (i, 0)),
            pl.BlockSpec((ROWB, 16), lambda i: (i, 0)),
            pl.BlockSpec((ROWB, 16), lambda i: (i, 0)),
            pl.BlockSpec((ROWB, 16), lambda i: (i, 0)),
        ],
        out_shape=[
            jax.ShapeDtypeStruct((N_PAD, 16), jnp.float32),
            jax.ShapeDtypeStruct((N_PAD, 16), jnp.float32),
            jax.ShapeDtypeStruct((N_PAD, 16), jnp.float32),
            jax.ShapeDtypeStruct((N_PAD, 16), jnp.float32),
        ],
    )(acc1, h1p, exs1, den1, b1p, w2p, a2sm, a2dm)


def _tc3_body(acc_ref, h2_ref, exs2_ref, den2_ref, b2_ref, out_ref):
    coef = exs2_ref[...] / (den2_ref[...] + 1e-16)
    out_ref[...] = acc_ref[...] + h2_ref[...] * coef + b2_ref[...]


def _tc3(acc2, h2, exs2, den2, b2):
    return pl.pallas_call(
        _tc3_body,
        grid=(NBLK,),
        in_specs=[
            pl.BlockSpec((ROWB, 16), lambda i: (i, 0)),
            pl.BlockSpec((ROWB, 16), lambda i: (i, 0)),
            pl.BlockSpec((ROWB, 16), lambda i: (i, 0)),
            pl.BlockSpec((ROWB, 16), lambda i: (i, 0)),
            pl.BlockSpec((1, 16), lambda i: (0, 0)),
        ],
        out_specs=pl.BlockSpec((ROWB, 16), lambda i: (i, 0)),
        out_shape=jax.ShapeDtypeStruct((N_PAD, 16), jnp.float32),
    )(acc2, h2, exs2, den2, b2)


_PERM = np.arange(64).reshape(HEADS, HID).T.reshape(-1)
_MASK16 = (np.arange(64)[:, None] % 8 == np.arange(16)[None, :] % 8)
_MASK16 = _MASK16.astype(np.float32)


def kernel(x, edge_index, W1, att_src1, att_dst1, b1, W2, att_src2, att_dst2, b2):
    f32 = jnp.float32
    x_pad = jnp.zeros((N_PAD, IN_CH), f32).at[:N].set(x)
    w1p = W1[:, _PERM]
    b1p = b1[_PERM].reshape(1, 64)
    w2p = W2[_PERM, :]
    att_s1p = att_src1[0].T.reshape(64)
    att_d1p = att_dst1[0].T.reshape(64)
    asm = att_s1p[:, None] * _MASK16
    adm = att_d1p[:, None] * _MASK16
    a2sm = jnp.broadcast_to(att_src2[0, 0][:, None], (16, 16)).astype(f32)
    a2dm = jnp.broadcast_to(att_dst2[0, 0][:, None], (16, 16)).astype(f32)

    src = edge_index[0].astype(jnp.int32)
    dst = edge_index[1].astype(jnp.int32)
    padv = jnp.full((E_PAD - E,), N, jnp.int32)
    srcJ = jnp.concatenate([src, padv]).reshape(NW, NCHUNK, CH)
    dstJ = jnp.concatenate([dst, padv]).reshape(NW, NCHUNK, CH)

    z16 = jnp.zeros((STRIPE, 16), f32)
    z64 = jnp.zeros((STRIPE, 64), f32)

    h1p, as16, ad16, exs1 = _tc1(x_pad, w1p, asm, adm)
    ex1, denp1 = _edge_scores_kernel()(as16, ad16, srcJ, dstJ, z16)
    den1 = denp1[0] + denp1[1] + exs1
    (accp1,) = _aggregate(64)(h1p, den1, ex1, srcJ, dstJ, z64)
    acc1 = accp1[0] + accp1[1]

    h2, a2s16, a2d16, exs2 = _tc2(acc1, h1p, exs1, den1, b1p, w2p, a2sm, a2dm)
    ex2, denp2 = _edge_scores_kernel()(a2s16, a2d16, srcJ, dstJ, z16)
    den2 = denp2[0] + denp2[1] + exs2
    (accp2,) = _aggregate(16)(h2, den2, ex2, srcJ, dstJ, z16)
    acc2 = accp2[0] + accp2[1]

    out = _tc3(acc2, h2, exs2, den2, b2.reshape(1, 16).astype(f32))
    return out[:N]


# speedup vs baseline: 84.6400x; 2.5498x over previous
import functools

import jax
import jax.numpy as jnp
import numpy as np
from jax import lax
from jax.experimental import pallas as pl
from jax.experimental.pallas import tpu as pltpu, tpu_sc as plsc

N = 10000
E = 320000
IN_CH = 128
HID = 8
HEADS = 8
OUT_CH = 16

NC, NS = 2, 16
NW = NC * NS
N_PAD = 10240
STRIPE = N_PAD // NS
EPT = 10240
E_PAD = EPT * NW
CH = 128
NCHUNK = EPT // CH
ROWB = 256
NBLK = N_PAD // ROWB

_sc_params = pltpu.CompilerParams(use_tc_tiling_on_sc=False)


@functools.lru_cache(maxsize=None)
def _sc_mesh():
    return plsc.VectorSubcoreMesh(core_axis_name="c", subcore_axis_name="s")


def _make_edge_pass(D):
    DV = D // 16

    def body(asrc_hbm, adst_hbm, h_hbm, src_hbm, dst_hbm, z16_hbm, zD_hbm,
             den_hbm, acc_hbm,
             src_v, dst_v, as0_v, ad0_v, h0_v, as1_v, ad1_v, h1_v, ex_v,
             den_sh, acc_sh, sa0, sh0, sa1, sh1):
        cid = lax.axis_index("c")
        sid = lax.axis_index("s")
        wid = sid * NC + cid

        pltpu.sync_copy(src_hbm.at[wid], src_v)
        pltpu.sync_copy(dst_hbm.at[wid], dst_v)
        pltpu.sync_copy(z16_hbm, den_sh.at[pl.ds(sid * STRIPE, STRIPE)])
        pltpu.sync_copy(zD_hbm, acc_sh.at[pl.ds(sid * STRIPE, STRIPE)])
        plsc.subcore_barrier()

        bufs = [(as0_v, ad0_v, h0_v, sa0, sh0), (as1_v, ad1_v, h1_v, sa1, sh1)]
        pend = {}

        def prefetch(j, b):
            as_v, ad_v, h_v, sem_a, sem_h = bufs[b]
            pend[b] = (
                pltpu.async_copy(asrc_hbm.at[src_v.at[j]], as_v, sem_a),
                pltpu.async_copy(adst_hbm.at[dst_v.at[j]], ad_v, sem_a),
                pltpu.async_copy(h_hbm.at[src_v.at[j]], h_v, sem_h),
            )

        prefetch(0, 0)
        for j in range(NCHUNK):
            b = j % 2
            if j + 1 < NCHUNK:
                prefetch(j + 1, 1 - b)
            as_v, ad_v, h_v, _, _ = bufs[b]
            ca, cb, chh = pend[b]
            ca.wait()
            cb.wait()

            def eloop(i, _):
                s = as_v[i, :] + ad_v[i, :]
                ex_v[i, :] = jnp.exp(jnp.maximum(s, 0.2 * s))
                return ()
            lax.fori_loop(0, CH, eloop, ())
            pltpu.sync_copy(ex_v, den_sh.at[dst_v.at[j]], add=True)

            chh.wait()

            def mloop(i, _):
                c = ex_v[i, :]
                for q in range(DV):
                    h_v[i, pl.ds(q * 16, 16)] = h_v[i, pl.ds(q * 16, 16)] * c
                return ()
            lax.fori_loop(0, CH, mloop, ())
            pltpu.sync_copy(h_v, acc_sh.at[dst_v.at[j]], add=True)

        plsc.subcore_barrier()
        pltpu.sync_copy(den_sh.at[pl.ds(sid * STRIPE, STRIPE)],
                        den_hbm.at[cid, pl.ds(sid * STRIPE, STRIPE)])
        pltpu.sync_copy(acc_sh.at[pl.ds(sid * STRIPE, STRIPE)],
                        acc_hbm.at[cid, pl.ds(sid * STRIPE, STRIPE)])

    return pl.kernel(
        body,
        out_type=[
            jax.ShapeDtypeStruct((NC, N_PAD, 16), jnp.float32),
            jax.ShapeDtypeStruct((NC, N_PAD, D), jnp.float32),
        ],
        mesh=_sc_mesh(),
        scratch_types=[
            pltpu.VMEM((NCHUNK, CH), jnp.int32),
            pltpu.VMEM((NCHUNK, CH), jnp.int32),
            pltpu.VMEM((CH, 16), jnp.float32),
            pltpu.VMEM((CH, 16), jnp.float32),
            pltpu.VMEM((CH, D), jnp.float32),
            pltpu.VMEM((CH, 16), jnp.float32),
            pltpu.VMEM((CH, 16), jnp.float32),
            pltpu.VMEM((CH, D), jnp.float32),
            pltpu.VMEM((CH, 16), jnp.float32),
            pltpu.VMEM_SHARED((N_PAD, 16), jnp.float32),
            pltpu.VMEM_SHARED((N_PAD, D), jnp.float32),
            pltpu.SemaphoreType.DMA,
            pltpu.SemaphoreType.DMA,
            pltpu.SemaphoreType.DMA,
            pltpu.SemaphoreType.DMA,
        ],
        compiler_params=_sc_params,
    )


_edge_pass = functools.lru_cache(maxsize=None)(_make_edge_pass)


def _tc1_body(x_ref, w1_ref, asm_ref, adm_ref,
              h1_ref, as16_ref, ad16_ref, exs_ref):
    h1 = jnp.dot(x_ref[...], w1_ref[...], preferred_element_type=jnp.float32)
    h1_ref[...] = h1
    a_s = jnp.dot(h1, asm_ref[...], preferred_element_type=jnp.float32)
    a_d = jnp.dot(h1, adm_ref[...], preferred_element_type=jnp.float32)
    as16_ref[...] = a_s
    ad16_ref[...] = a_d
    s = a_s + a_d
    exs_ref[...] = jnp.exp(jnp.maximum(s, 0.2 * s))


def _tc1(x_pad, w1p, asm, adm):
    return pl.pallas_call(
        _tc1_body,
        grid=(NBLK,),
        in_specs=[
            pl.BlockSpec((ROWB, IN_CH), lambda i: (i, 0)),
            pl.BlockSpec((IN_CH, 64), lambda i: (0, 0)),
            pl.BlockSpec((64, 16), lambda i: (0, 0)),
            pl.BlockSpec((64, 16), lambda i: (0, 0)),
        ],
        out_specs=[
            pl.BlockSpec((ROWB, 64), lambda i: (i, 0)),
            pl.BlockSpec((ROWB, 16), lambda i: (i, 0)),
            pl.BlockSpec((ROWB, 16), lambda i: (i, 0)),
            pl.BlockSpec((ROWB, 16), lambda i: (i, 0)),
        ],
        out_shape=[
            jax.ShapeDtypeStruct((N_PAD, 64), jnp.float32),
            jax.ShapeDtypeStruct((N_PAD, 16), jnp.float32),
            jax.ShapeDtypeStruct((N_PAD, 16), jnp.float32),
            jax.ShapeDtypeStruct((N_PAD, 16), jnp.float32),
        ],
    )(x_pad, w1p, asm, adm)


def _tc2_body(denp_ref, accp_ref, h1_ref, exs_ref,
              b1_ref, w2_ref, a2sm_ref, a2dm_ref,
              h2_ref, a2s_ref, a2d_ref, exs2_ref):
    exs = exs_ref[...]
    den = denp_ref[0] + denp_ref[1] + exs
    num16 = h1_ref[...]
    coef = exs / (den + 1e-16)
    rden = 1.0 / (den + 1e-16)
    coef64 = jnp.concatenate([coef, coef, coef, coef], axis=1)
    rden64 = jnp.concatenate([rden, rden, rden, rden], axis=1)
    acc = accp_ref[0] + accp_ref[1]
    out1 = acc * rden64 + num16 * coef64 + b1_ref[...]
    hin2 = jnp.where(out1 > 0, out1, jnp.exp(jnp.minimum(out1, 0.0)) - 1.0)
    h2 = jnp.dot(hin2, w2_ref[...], preferred_element_type=jnp.float32)
    h2_ref[...] = h2
    a2s = jnp.dot(h2, a2sm_ref[...], preferred_element_type=jnp.float32)
    a2d = jnp.dot(h2, a2dm_ref[...], preferred_element_type=jnp.float32)
    a2s_ref[...] = a2s
    a2d_ref[...] = a2d
    s = a2s + a2d
    exs2_ref[...] = jnp.exp(jnp.maximum(s, 0.2 * s))


def _tc2(denp1, accp1, h1p, exs1, b1p, w2p, a2sm, a2dm):
    return pl.pallas_call(
        _tc2_body,
        grid=(NBLK,),
        in_specs=[
            pl.BlockSpec((NC, ROWB, 16), lambda i: (0, i, 0)),
            pl.BlockSpec((NC, ROWB, 64), lambda i: (0, i, 0)),
            pl.BlockSpec((ROWB, 64), lambda i: (i, 0)),
            pl.BlockSpec((ROWB, 16), lambda i: (i, 0)),
            pl.BlockSpec((1, 64), lambda i: (0, 0)),
            pl.BlockSpec((64, 16), lambda i: (0, 0)),
            pl.BlockSpec((16, 16), lambda i: (0, 0)),
            pl.BlockSpec((16, 16), lambda i: (0, 0)),
        ],
        out_specs=[
            pl.BlockSpec((ROWB, 16), lambda i: (i, 0)),
            pl.BlockSpec((ROWB, 16), lambda i: (i, 0)),
            pl.BlockSpec((ROWB, 16), lambda i: (i, 0)),
            pl.BlockSpec((ROWB, 16), lambda i: (i, 0)),
        ],
        out_shape=[
            jax.ShapeDtypeStruct((N_PAD, 16), jnp.float32),
            jax.ShapeDtypeStruct((N_PAD, 16), jnp.float32),
            jax.ShapeDtypeStruct((N_PAD, 16), jnp.float32),
            jax.ShapeDtypeStruct((N_PAD, 16), jnp.float32),
        ],
    )(denp1, accp1, h1p, exs1, b1p, w2p, a2sm, a2dm)


def _tc3_body(denp_ref, accp_ref, h2_ref, exs2_ref,
              b2_ref, out_ref):
    exs2 = exs2_ref[...]
    den = denp_ref[0] + denp_ref[1] + exs2
    rden = 1.0 / (den + 1e-16)
    acc = accp_ref[0] + accp_ref[1]
    out_ref[...] = acc * rden + h2_ref[...] * (exs2 * rden) + b2_ref[...]


def _tc3(denp2, accp2, h2, exs2, b2):
    return pl.pallas_call(
        _tc3_body,
        grid=(NBLK,),
        in_specs=[
            pl.BlockSpec((NC, ROWB, 16), lambda i: (0, i, 0)),
            pl.BlockSpec((NC, ROWB, 16), lambda i: (0, i, 0)),
            pl.BlockSpec((ROWB, 16), lambda i: (i, 0)),
            pl.BlockSpec((ROWB, 16), lambda i: (i, 0)),
            pl.BlockSpec((1, 16), lambda i: (0, 0)),
        ],
        out_specs=pl.BlockSpec((ROWB, 16), lambda i: (i, 0)),
        out_shape=jax.ShapeDtypeStruct((N_PAD, 16), jnp.float32),
    )(denp2, accp2, h2, exs2, b2)


_PERM = np.arange(64).reshape(HEADS, HID).T.reshape(-1)
_MASK16 = (np.arange(64)[:, None] % 8 == np.arange(16)[None, :] % 8)
_MASK16 = _MASK16.astype(np.float32)


def kernel(x, edge_index, W1, att_src1, att_dst1, b1, W2, att_src2, att_dst2, b2):
    f32 = jnp.float32
    x_pad = jnp.zeros((N_PAD, IN_CH), f32).at[:N].set(x)
    w1p = W1[:, _PERM]
    b1p = b1[_PERM].reshape(1, 64)
    w2p = W2[_PERM, :]
    att_s1p = att_src1[0].T.reshape(64)
    att_d1p = att_dst1[0].T.reshape(64)
    asm = att_s1p[:, None] * _MASK16
    adm = att_d1p[:, None] * _MASK16
    a2sm = jnp.broadcast_to(att_src2[0, 0][:, None], (16, 16)).astype(f32)
    a2dm = jnp.broadcast_to(att_dst2[0, 0][:, None], (16, 16)).astype(f32)

    src = edge_index[0].astype(jnp.int32)
    dst = edge_index[1].astype(jnp.int32)
    padv = jnp.full((E_PAD - E,), N, jnp.int32)
    srcJ = jnp.concatenate([src, padv]).reshape(NW, NCHUNK, CH)
    dstJ = jnp.concatenate([dst, padv]).reshape(NW, NCHUNK, CH)

    z16 = jnp.zeros((STRIPE, 16), f32)
    z64 = jnp.zeros((STRIPE, 64), f32)

    h1p, as16, ad16, exs1 = _tc1(x_pad, w1p, asm, adm)
    denp1, accp1 = _edge_pass(64)(as16, ad16, h1p, srcJ, dstJ, z16, z64)
    h2, a2s16, a2d16, exs2 = _tc2(denp1, accp1, h1p, exs1, b1p, w2p, a2sm, a2dm)

    denp2, accp2 = _edge_pass(16)(a2s16, a2d16, h2, srcJ, dstJ, z16, z16)
    out = _tc3(denp2, accp2, h2, exs2, b2.reshape(1, 16).astype(f32))
    return out[:N]


# speedup vs baseline: 88.5448x; 1.0461x over previous
import functools

import jax
import jax.numpy as jnp
import numpy as np
from jax import lax
from jax.experimental import pallas as pl
from jax.experimental.pallas import tpu as pltpu, tpu_sc as plsc

N = 10000
E = 320000
IN_CH = 128
HID = 8
HEADS = 8
OUT_CH = 16

NC, NS = 2, 16
NW = NC * NS
N_PAD = 10240
STRIPE = N_PAD // NS
EPT = 10240
E_PAD = EPT * NW
CH = 128
NCHUNK = EPT // CH
ROWB = 256
NBLK = N_PAD // ROWB

_sc_params = pltpu.CompilerParams(use_tc_tiling_on_sc=False)


@functools.lru_cache(maxsize=None)
def _sc_mesh():
    return plsc.VectorSubcoreMesh(core_axis_name="c", subcore_axis_name="s")


def _make_edge_pass(D):
    DV = D // 16
    NB = 3

    def body(asrc_hbm, adst_hbm, h_hbm, src_hbm, dst_hbm, z16_hbm, zD_hbm,
             den_hbm, acc_hbm,
             src_v, dst_v,
             as0_v, ad0_v, h0_v, ex0_v, as1_v, ad1_v, h1_v, ex1_v,
             as2_v, ad2_v, h2_v, ex2_v,
             den_sh, acc_sh,
             sa0, sh0, sw0, sa1, sh1, sw1, sa2, sh2, sw2):
        cid = lax.axis_index("c")
        sid = lax.axis_index("s")
        wid = sid * NC + cid

        pltpu.sync_copy(src_hbm.at[wid], src_v)
        pltpu.sync_copy(dst_hbm.at[wid], dst_v)
        pltpu.sync_copy(z16_hbm, den_sh.at[pl.ds(sid * STRIPE, STRIPE)])
        pltpu.sync_copy(zD_hbm, acc_sh.at[pl.ds(sid * STRIPE, STRIPE)])
        plsc.subcore_barrier()

        bufs = [(as0_v, ad0_v, h0_v, ex0_v, sa0, sh0, sw0),
                (as1_v, ad1_v, h1_v, ex1_v, sa1, sh1, sw1),
                (as2_v, ad2_v, h2_v, ex2_v, sa2, sh2, sw2)]
        pend_g = {}
        pend_w = {}

        def prefetch(j, b):
            as_v, ad_v, h_v, _, sem_a, sem_h, _ = bufs[b]
            pend_g[b] = (
                pltpu.async_copy(asrc_hbm.at[src_v.at[j]], as_v, sem_a),
                pltpu.async_copy(adst_hbm.at[dst_v.at[j]], ad_v, sem_a),
                pltpu.async_copy(h_hbm.at[src_v.at[j]], h_v, sem_h),
            )

        prefetch(0, 0)
        for j in range(NCHUNK):
            b = j % NB
            nb = (j + 1) % NB
            if j + 1 < NCHUNK:
                if nb in pend_w:
                    for w in pend_w.pop(nb):
                        w.wait()
                prefetch(j + 1, nb)
            as_v, ad_v, h_v, ex_v, _, _, sem_w = bufs[b]
            ca, cb, chh = pend_g[b]
            ca.wait()
            cb.wait()
            chh.wait()

            def floop(i, _):
                s = as_v[i, :] + ad_v[i, :]
                c = jnp.exp(jnp.maximum(s, 0.2 * s))
                ex_v[i, :] = c
                for q in range(DV):
                    h_v[i, pl.ds(q * 16, 16)] = h_v[i, pl.ds(q * 16, 16)] * c
                return ()
            lax.fori_loop(0, CH, floop, ())

            pend_w[b] = (
                pltpu.async_copy(ex_v, den_sh.at[dst_v.at[j]], sem_w, add=True),
                pltpu.async_copy(h_v, acc_sh.at[dst_v.at[j]], sem_w, add=True),
            )

        for b in list(pend_w):
            for w in pend_w.pop(b):
                w.wait()
        plsc.subcore_barrier()
        pltpu.sync_copy(den_sh.at[pl.ds(sid * STRIPE, STRIPE)],
                        den_hbm.at[cid, pl.ds(sid * STRIPE, STRIPE)])
        pltpu.sync_copy(acc_sh.at[pl.ds(sid * STRIPE, STRIPE)],
                        acc_hbm.at[cid, pl.ds(sid * STRIPE, STRIPE)])

    vm = pltpu.VMEM
    return pl.kernel(
        body,
        out_type=[
            jax.ShapeDtypeStruct((NC, N_PAD, 16), jnp.float32),
            jax.ShapeDtypeStruct((NC, N_PAD, D), jnp.float32),
        ],
        mesh=_sc_mesh(),
        scratch_types=[
            vm((NCHUNK, CH), jnp.int32),
            vm((NCHUNK, CH), jnp.int32),
            vm((CH, 16), jnp.float32), vm((CH, 16), jnp.float32),
            vm((CH, D), jnp.float32), vm((CH, 16), jnp.float32),
            vm((CH, 16), jnp.float32), vm((CH, 16), jnp.float32),
            vm((CH, D), jnp.float32), vm((CH, 16), jnp.float32),
            vm((CH, 16), jnp.float32), vm((CH, 16), jnp.float32),
            vm((CH, D), jnp.float32), vm((CH, 16), jnp.float32),
            pltpu.VMEM_SHARED((N_PAD, 16), jnp.float32),
            pltpu.VMEM_SHARED((N_PAD, D), jnp.float32),
            pltpu.SemaphoreType.DMA, pltpu.SemaphoreType.DMA,
            pltpu.SemaphoreType.DMA, pltpu.SemaphoreType.DMA,
            pltpu.SemaphoreType.DMA, pltpu.SemaphoreType.DMA,
            pltpu.SemaphoreType.DMA, pltpu.SemaphoreType.DMA,
            pltpu.SemaphoreType.DMA,
        ],
        compiler_params=_sc_params,
    )


_edge_pass = functools.lru_cache(maxsize=None)(_make_edge_pass)


def _tc1_body(x_ref, w1_ref, asm_ref, adm_ref,
              h1_ref, as16_ref, ad16_ref, exs_ref):
    h1 = jnp.dot(x_ref[...], w1_ref[...], preferred_element_type=jnp.float32)
    h1_ref[...] = h1
    a_s = jnp.dot(h1, asm_ref[...], preferred_element_type=jnp.float32)
    a_d = jnp.dot(h1, adm_ref[...], preferred_element_type=jnp.float32)
    as16_ref[...] = a_s
    ad16_ref[...] = a_d
    s = a_s + a_d
    exs_ref[...] = jnp.exp(jnp.maximum(s, 0.2 * s))


def _tc1(x_pad, w1p, asm, adm):
    return pl.pallas_call(
        _tc1_body,
        grid=(NBLK,),
        in_specs=[
            pl.BlockSpec((ROWB, IN_CH), lambda i: (i, 0)),
            pl.BlockSpec((IN_CH, 64), lambda i: (0, 0)),
            pl.BlockSpec((64, 16), lambda i: (0, 0)),
            pl.BlockSpec((64, 16), lambda i: (0, 0)),
        ],
        out_specs=[
            pl.BlockSpec((ROWB, 64), lambda i: (i, 0)),
            pl.BlockSpec((ROWB, 16), lambda i: (i, 0)),
            pl.BlockSpec((ROWB, 16), lambda i: (i, 0)),
            pl.BlockSpec((ROWB, 16), lambda i: (i, 0)),
        ],
        out_shape=[
            jax.ShapeDtypeStruct((N_PAD, 64), jnp.float32),
            jax.ShapeDtypeStruct((N_PAD, 16), jnp.float32),
            jax.ShapeDtypeStruct((N_PAD, 16), jnp.float32),
            jax.ShapeDtypeStruct((N_PAD, 16), jnp.float32),
        ],
    )(x_pad, w1p, asm, adm)


def _tc2_body(denp_ref, accp_ref, h1_ref, exs_ref,
              b1_ref, w2_ref, a2sm_ref, a2dm_ref,
              h2_ref, a2s_ref, a2d_ref, exs2_ref):
    exs = exs_ref[...]
    den = denp_ref[0] + denp_ref[1] + exs
    num16 = h1_ref[...]
    coef = exs / (den + 1e-16)
    rden = 1.0 / (den + 1e-16)
    coef64 = jnp.concatenate([coef, coef, coef, coef], axis=1)
    rden64 = jnp.concatenate([rden, rden, rden, rden], axis=1)
    acc = accp_ref[0] + accp_ref[1]
    out1 = acc * rden64 + num16 * coef64 + b1_ref[...]
    hin2 = jnp.where(out1 > 0, out1, jnp.exp(jnp.minimum(out1, 0.0)) - 1.0)
    h2 = jnp.dot(hin2, w2_ref[...], preferred_element_type=jnp.float32)
    h2_ref[...] = h2
    a2s = jnp.dot(h2, a2sm_ref[...], preferred_element_type=jnp.float32)
    a2d = jnp.dot(h2, a2dm_ref[...], preferred_element_type=jnp.float32)
    a2s_ref[...] = a2s
    a2d_ref[...] = a2d
    s = a2s + a2d
    exs2_ref[...] = jnp.exp(jnp.maximum(s, 0.2 * s))


def _tc2(denp1, accp1, h1p, exs1, b1p, w2p, a2sm, a2dm):
    return pl.pallas_call(
        _tc2_body,
        grid=(NBLK,),
        in_specs=[
            pl.BlockSpec((NC, ROWB, 16), lambda i: (0, i, 0)),
            pl.BlockSpec((NC, ROWB, 64), lambda i: (0, i, 0)),
            pl.BlockSpec((ROWB, 64), lambda i: (i, 0)),
            pl.BlockSpec((ROWB, 16), lambda i: (i, 0)),
            pl.BlockSpec((1, 64), lambda i: (0, 0)),
            pl.BlockSpec((64, 16), lambda i: (0, 0)),
            pl.BlockSpec((16, 16), lambda i: (0, 0)),
            pl.BlockSpec((16, 16), lambda i: (0, 0)),
        ],
        out_specs=[
            pl.BlockSpec((ROWB, 16), lambda i: (i, 0)),
            pl.BlockSpec((ROWB, 16), lambda i: (i, 0)),
            pl.BlockSpec((ROWB, 16), lambda i: (i, 0)),
            pl.BlockSpec((ROWB, 16), lambda i: (i, 0)),
        ],
        out_shape=[
            jax.ShapeDtypeStruct((N_PAD, 16), jnp.float32),
            jax.ShapeDtypeStruct((N_PAD, 16), jnp.float32),
            jax.ShapeDtypeStruct((N_PAD, 16), jnp.float32),
            jax.ShapeDtypeStruct((N_PAD, 16), jnp.float32),
        ],
    )(denp1, accp1, h1p, exs1, b1p, w2p, a2sm, a2dm)


def _tc3_body(denp_ref, accp_ref, h2_ref, exs2_ref,
              b2_ref, out_ref):
    exs2 = exs2_ref[...]
    den = denp_ref[0] + denp_ref[1] + exs2
    rden = 1.0 / (den + 1e-16)
    acc = accp_ref[0] + accp_ref[1]
    out_ref[...] = acc * rden + h2_ref[...] * (exs2 * rden) + b2_ref[...]


def _tc3(denp2, accp2, h2, exs2, b2):
    return pl.pallas_call(
        _tc3_body,
        grid=(NBLK,),
        in_specs=[
            pl.BlockSpec((NC, ROWB, 16), lambda i: (0, i, 0)),
            pl.BlockSpec((NC, ROWB, 16), lambda i: (0, i, 0)),
            pl.BlockSpec((ROWB, 16), lambda i: (i, 0)),
            pl.BlockSpec((ROWB, 16), lambda i: (i, 0)),
            pl.BlockSpec((1, 16), lambda i: (0, 0)),
        ],
        out_specs=pl.BlockSpec((ROWB, 16), lambda i: (i, 0)),
        out_shape=jax.ShapeDtypeStruct((N_PAD, 16), jnp.float32),
    )(denp2, accp2, h2, exs2, b2)


_PERM = np.arange(64).reshape(HEADS, HID).T.reshape(-1)
_MASK16 = (np.arange(64)[:, None] % 8 == np.arange(16)[None, :] % 8)
_MASK16 = _MASK16.astype(np.float32)


def kernel(x, edge_index, W1, att_src1, att_dst1, b1, W2, att_src2, att_dst2, b2):
    f32 = jnp.float32
    x_pad = jnp.zeros((N_PAD, IN_CH), f32).at[:N].set(x)
    w1p = W1[:, _PERM]
    b1p = b1[_PERM].reshape(1, 64)
    w2p = W2[_PERM, :]
    att_s1p = att_src1[0].T.reshape(64)
    att_d1p = att_dst1[0].T.reshape(64)
    asm = att_s1p[:, None] * _MASK16
    adm = att_d1p[:, None] * _MASK16
    a2sm = jnp.broadcast_to(att_src2[0, 0][:, None], (16, 16)).astype(f32)
    a2dm = jnp.broadcast_to(att_dst2[0, 0][:, None], (16, 16)).astype(f32)

    src = edge_index[0].astype(jnp.int32)
    dst = edge_index[1].astype(jnp.int32)
    padv = jnp.full((E_PAD - E,), N, jnp.int32)
    srcJ = jnp.concatenate([src, padv]).reshape(NW, NCHUNK, CH)
    dstJ = jnp.concatenate([dst, padv]).reshape(NW, NCHUNK, CH)

    z16 = jnp.zeros((STRIPE, 16), f32)
    z64 = jnp.zeros((STRIPE, 64), f32)

    h1p, as16, ad16, exs1 = _tc1(x_pad, w1p, asm, adm)
    denp1, accp1 = _edge_pass(64)(as16, ad16, h1p, srcJ, dstJ, z16, z64)
    h2, a2s16, a2d16, exs2 = _tc2(denp1, accp1, h1p, exs1, b1p, w2p, a2sm, a2dm)

    denp2, accp2 = _edge_pass(16)(a2s16, a2d16, h2, srcJ, dstJ, z16, z16)
    out = _tc3(denp2, accp2, h2, exs2, b2.reshape(1, 16).astype(f32))
    return out[:N]


# speedup vs baseline: 92.6037x; 1.0458x over previous
import functools

import jax
import jax.numpy as jnp
import numpy as np
from jax import lax
from jax.experimental import pallas as pl
from jax.experimental.pallas import tpu as pltpu, tpu_sc as plsc

N = 10000
E = 320000
IN_CH = 128
HID = 8
HEADS = 8
OUT_CH = 16

NC, NS = 2, 16
NW = NC * NS
N_PAD = 10240
STRIPE = N_PAD // NS
EPT = 10240
E_PAD = EPT * NW
CH = 128
NCHUNK = EPT // CH
ROWB = 256
NBLK = N_PAD // ROWB

_sc_params = pltpu.CompilerParams(use_tc_tiling_on_sc=False)


@functools.lru_cache(maxsize=None)
def _sc_mesh():
    return plsc.VectorSubcoreMesh(core_axis_name="c", subcore_axis_name="s")


def _make_edge_pass(D):
    DV = D // 16
    NB = 3

    def body(asrc_hbm, adst_hbm, h_hbm, src_hbm, dst_hbm, z16_hbm, zD_hbm,
             den_hbm, acc_hbm,
             src_v, dst_v,
             as0_v, ad0_v, h0_v, ex0_v, as1_v, ad1_v, h1_v, ex1_v,
             as2_v, ad2_v, h2_v, ex2_v,
             den_sh, acc_sh,
             sa0, sh0, sw0, sa1, sh1, sw1, sa2, sh2, sw2):
        cid = lax.axis_index("c")
        sid = lax.axis_index("s")
        wid = sid * NC + cid

        pltpu.sync_copy(src_hbm.at[wid], src_v)
        pltpu.sync_copy(dst_hbm.at[wid], dst_v)
        pltpu.sync_copy(z16_hbm, den_sh.at[pl.ds(sid * STRIPE, STRIPE)])
        pltpu.sync_copy(zD_hbm, acc_sh.at[pl.ds(sid * STRIPE, STRIPE)])
        plsc.subcore_barrier()

        bufs = [(as0_v, ad0_v, h0_v, ex0_v, sa0, sh0, sw0),
                (as1_v, ad1_v, h1_v, ex1_v, sa1, sh1, sw1),
                (as2_v, ad2_v, h2_v, ex2_v, sa2, sh2, sw2)]
        pend_g = {}
        pend_w = {}

        def prefetch(j, b):
            as_v, ad_v, h_v, _, sem_a, sem_h, _ = bufs[b]
            pend_g[b] = (
                pltpu.async_copy(asrc_hbm.at[src_v.at[j]], as_v, sem_a),
                pltpu.async_copy(adst_hbm.at[dst_v.at[j]], ad_v, sem_a),
                pltpu.async_copy(h_hbm.at[src_v.at[j]], h_v, sem_h),
            )

        prefetch(0, 0)
        for j in range(NCHUNK):
            b = j % NB
            nb = (j + 1) % NB
            if j + 1 < NCHUNK:
                if nb in pend_w:
                    for w in pend_w.pop(nb):
                        w.wait()
                prefetch(j + 1, nb)
            as_v, ad_v, h_v, ex_v, _, _, sem_w = bufs[b]
            ca, cb, chh = pend_g[b]
            ca.wait()
            cb.wait()
            chh.wait()

            @plsc.parallel_loop(0, CH, unroll=1)
            def floop(i):
                s = as_v[i, :] + ad_v[i, :]
                c = jnp.exp(jnp.maximum(s, 0.2 * s))
                ex_v[i, :] = c
                for q in range(DV):
                    h_v[i, pl.ds(q * 16, 16)] = h_v[i, pl.ds(q * 16, 16)] * c

            pend_w[b] = (
                pltpu.async_copy(ex_v, den_sh.at[dst_v.at[j]], sem_w, add=True),
                pltpu.async_copy(h_v, acc_sh.at[dst_v.at[j]], sem_w, add=True),
            )

        for b in list(pend_w):
            for w in pend_w.pop(b):
                w.wait()
        plsc.subcore_barrier()
        pltpu.sync_copy(den_sh.at[pl.ds(sid * STRIPE, STRIPE)],
                        den_hbm.at[cid, pl.ds(sid * STRIPE, STRIPE)])
        pltpu.sync_copy(acc_sh.at[pl.ds(sid * STRIPE, STRIPE)],
                        acc_hbm.at[cid, pl.ds(sid * STRIPE, STRIPE)])

    vm = pltpu.VMEM
    return pl.kernel(
        body,
        out_type=[
            jax.ShapeDtypeStruct((NC, N_PAD, 16), jnp.float32),
            jax.ShapeDtypeStruct((NC, N_PAD, D), jnp.float32),
        ],
        mesh=_sc_mesh(),
        scratch_types=[
            vm((NCHUNK, CH), jnp.int32),
            vm((NCHUNK, CH), jnp.int32),
            vm((CH, 16), jnp.float32), vm((CH, 16), jnp.float32),
            vm((CH, D), jnp.float32), vm((CH, 16), jnp.float32),
            vm((CH, 16), jnp.float32), vm((CH, 16), jnp.float32),
            vm((CH, D), jnp.float32), vm((CH, 16), jnp.float32),
            vm((CH, 16), jnp.float32), vm((CH, 16), jnp.float32),
            vm((CH, D), jnp.float32), vm((CH, 16), jnp.float32),
            pltpu.VMEM_SHARED((N_PAD, 16), jnp.float32),
            pltpu.VMEM_SHARED((N_PAD, D), jnp.float32),
            pltpu.SemaphoreType.DMA, pltpu.SemaphoreType.DMA,
            pltpu.SemaphoreType.DMA, pltpu.SemaphoreType.DMA,
            pltpu.SemaphoreType.DMA, pltpu.SemaphoreType.DMA,
            pltpu.SemaphoreType.DMA, pltpu.SemaphoreType.DMA,
            pltpu.SemaphoreType.DMA,
        ],
        compiler_params=_sc_params,
    )


_edge_pass = functools.lru_cache(maxsize=None)(_make_edge_pass)


def _tc1_body(x_ref, w1_ref, asm_ref, adm_ref,
              h1_ref, as16_ref, ad16_ref, exs_ref):
    h1 = jnp.dot(x_ref[...], w1_ref[...], preferred_element_type=jnp.float32)
    h1_ref[...] = h1
    a_s = jnp.dot(h1, asm_ref[...], preferred_element_type=jnp.float32)
    a_d = jnp.dot(h1, adm_ref[...], preferred_element_type=jnp.float32)
    as16_ref[...] = a_s
    ad16_ref[...] = a_d
    s = a_s + a_d
    exs_ref[...] = jnp.exp(jnp.maximum(s, 0.2 * s))


def _tc1(x_pad, w1p, asm, adm):
    return pl.pallas_call(
        _tc1_body,
        grid=(NBLK,),
        in_specs=[
            pl.BlockSpec((ROWB, IN_CH), lambda i: (i, 0)),
            pl.BlockSpec((IN_CH, 64), lambda i: (0, 0)),
            pl.BlockSpec((64, 16), lambda i: (0, 0)),
            pl.BlockSpec((64, 16), lambda i: (0, 0)),
        ],
        out_specs=[
            pl.BlockSpec((ROWB, 64), lambda i: (i, 0)),
            pl.BlockSpec((ROWB, 16), lambda i: (i, 0)),
            pl.BlockSpec((ROWB, 16), lambda i: (i, 0)),
            pl.BlockSpec((ROWB, 16), lambda i: (i, 0)),
        ],
        out_shape=[
            jax.ShapeDtypeStruct((N_PAD, 64), jnp.float32),
            jax.ShapeDtypeStruct((N_PAD, 16), jnp.float32),
            jax.ShapeDtypeStruct((N_PAD, 16), jnp.float32),
            jax.ShapeDtypeStruct((N_PAD, 16), jnp.float32),
        ],
    )(x_pad, w1p, asm, adm)


def _tc2_body(denp_ref, accp_ref, h1_ref, exs_ref,
              b1_ref, w2_ref, a2sm_ref, a2dm_ref,
              h2_ref, a2s_ref, a2d_ref, exs2_ref):
    exs = exs_ref[...]
    den = denp_ref[0] + denp_ref[1] + exs
    num16 = h1_ref[...]
    coef = exs / (den + 1e-16)
    rden = 1.0 / (den + 1e-16)
    coef64 = jnp.concatenate([coef, coef, coef, coef], axis=1)
    rden64 = jnp.concatenate([rden, rden, rden, rden], axis=1)
    acc = accp_ref[0] + accp_ref[1]
    out1 = acc * rden64 + num16 * coef64 + b1_ref[...]
    hin2 = jnp.where(out1 > 0, out1, jnp.exp(jnp.minimum(out1, 0.0)) - 1.0)
    h2 = jnp.dot(hin2, w2_ref[...], preferred_element_type=jnp.float32)
    h2_ref[...] = h2
    a2s = jnp.dot(h2, a2sm_ref[...], preferred_element_type=jnp.float32)
    a2d = jnp.dot(h2, a2dm_ref[...], preferred_element_type=jnp.float32)
    a2s_ref[...] = a2s
    a2d_ref[...] = a2d
    s = a2s + a2d
    exs2_ref[...] = jnp.exp(jnp.maximum(s, 0.2 * s))


def _tc2(denp1, accp1, h1p, exs1, b1p, w2p, a2sm, a2dm):
    return pl.pallas_call(
        _tc2_body,
        grid=(NBLK,),
        in_specs=[
            pl.BlockSpec((NC, ROWB, 16), lambda i: (0, i, 0)),
            pl.BlockSpec((NC, ROWB, 64), lambda i: (0, i, 0)),
            pl.BlockSpec((ROWB, 64), lambda i: (i, 0)),
            pl.BlockSpec((ROWB, 16), lambda i: (i, 0)),
            pl.BlockSpec((1, 64), lambda i: (0, 0)),
            pl.BlockSpec((64, 16), lambda i: (0, 0)),
            pl.BlockSpec((16, 16), lambda i: (0, 0)),
            pl.BlockSpec((16, 16), lambda i: (0, 0)),
        ],
        out_specs=[
            pl.BlockSpec((ROWB, 16), lambda i: (i, 0)),
            pl.BlockSpec((ROWB, 16), lambda i: (i, 0)),
            pl.BlockSpec((ROWB, 16), lambda i: (i, 0)),
            pl.BlockSpec((ROWB, 16), lambda i: (i, 0)),
        ],
        out_shape=[
            jax.ShapeDtypeStruct((N_PAD, 16), jnp.float32),
            jax.ShapeDtypeStruct((N_PAD, 16), jnp.float32),
            jax.ShapeDtypeStruct((N_PAD, 16), jnp.float32),
            jax.ShapeDtypeStruct((N_PAD, 16), jnp.float32),
        ],
    )(denp1, accp1, h1p, exs1, b1p, w2p, a2sm, a2dm)


def _tc3_body(denp_ref, accp_ref, h2_ref, exs2_ref,
              b2_ref, out_ref):
    exs2 = exs2_ref[...]
    den = denp_ref[0] + denp_ref[1] + exs2
    rden = 1.0 / (den + 1e-16)
    acc = accp_ref[0] + accp_ref[1]
    out_ref[...] = acc * rden + h2_ref[...] * (exs2 * rden) + b2_ref[...]


def _tc3(denp2, accp2, h2, exs2, b2):
    return pl.pallas_call(
        _tc3_body,
        grid=(NBLK,),
        in_specs=[
            pl.BlockSpec((NC, ROWB, 16), lambda i: (0, i, 0)),
            pl.BlockSpec((NC, ROWB, 16), lambda i: (0, i, 0)),
            pl.BlockSpec((ROWB, 16), lambda i: (i, 0)),
            pl.BlockSpec((ROWB, 16), lambda i: (i, 0)),
            pl.BlockSpec((1, 16), lambda i: (0, 0)),
        ],
        out_specs=pl.BlockSpec((ROWB, 16), lambda i: (i, 0)),
        out_shape=jax.ShapeDtypeStruct((N_PAD, 16), jnp.float32),
    )(denp2, accp2, h2, exs2, b2)


_PERM = np.arange(64).reshape(HEADS, HID).T.reshape(-1)
_MASK16 = (np.arange(64)[:, None] % 8 == np.arange(16)[None, :] % 8)
_MASK16 = _MASK16.astype(np.float32)


def kernel(x, edge_index, W1, att_src1, att_dst1, b1, W2, att_src2, att_dst2, b2):
    f32 = jnp.float32
    x_pad = jnp.zeros((N_PAD, IN_CH), f32).at[:N].set(x)
    w1p = W1[:, _PERM]
    b1p = b1[_PERM].reshape(1, 64)
    w2p = W2[_PERM, :]
    att_s1p = att_src1[0].T.reshape(64)
    att_d1p = att_dst1[0].T.reshape(64)
    asm = att_s1p[:, None] * _MASK16
    adm = att_d1p[:, None] * _MASK16
    a2sm = jnp.broadcast_to(att_src2[0, 0][:, None], (16, 16)).astype(f32)
    a2dm = jnp.broadcast_to(att_dst2[0, 0][:, None], (16, 16)).astype(f32)

    src = edge_index[0].astype(jnp.int32)
    dst = edge_index[1].astype(jnp.int32)
    padv = jnp.full((E_PAD - E,), N, jnp.int32)
    srcJ = jnp.concatenate([src, padv]).reshape(NW, NCHUNK, CH)
    dstJ = jnp.concatenate([dst, padv]).reshape(NW, NCHUNK, CH)

    z16 = jnp.zeros((STRIPE, 16), f32)
    z64 = jnp.zeros((STRIPE, 64), f32)

    h1p, as16, ad16, exs1 = _tc1(x_pad, w1p, asm, adm)
    denp1, accp1 = _edge_pass(64)(as16, ad16, h1p, srcJ, dstJ, z16, z64)
    h2, a2s16, a2d16, exs2 = _tc2(denp1, accp1, h1p, exs1, b1p, w2p, a2sm, a2dm)

    denp2, accp2 = _edge_pass(16)(a2s16, a2d16, h2, srcJ, dstJ, z16, z16)
    out = _tc3(denp2, accp2, h2, exs2, b2.reshape(1, 16).astype(f32))
    return out[:N]


# speedup vs baseline: 93.1125x; 1.0055x over previous
import functools

import jax
import jax.numpy as jnp
import numpy as np
from jax import lax
from jax.experimental import pallas as pl
from jax.experimental.pallas import tpu as pltpu, tpu_sc as plsc

N = 10000
E = 320000
IN_CH = 128
HID = 8
HEADS = 8
OUT_CH = 16

NC, NS = 2, 16
NW = NC * NS
N_PAD = 10240
STRIPE = N_PAD // NS
EPT = 10240
E_PAD = EPT * NW
CH = 128
NCHUNK = EPT // CH
ROWB = 256
NBLK = N_PAD // ROWB

_sc_params = pltpu.CompilerParams(use_tc_tiling_on_sc=False)


@functools.lru_cache(maxsize=None)
def _sc_mesh():
    return plsc.VectorSubcoreMesh(core_axis_name="c", subcore_axis_name="s")


def _make_edge_pass(D):
    DV = D // 16
    NB = 3

    def body(asrc_hbm, adst_hbm, h_hbm, src_hbm, dst_hbm, z16_hbm, zD_hbm,
             den_hbm, acc_hbm,
             src_v, dst_v,
             as0_v, ad0_v, h0_v, ex0_v, as1_v, ad1_v, h1_v, ex1_v,
             as2_v, ad2_v, h2_v, ex2_v,
             den_sh, acc_sh,
             sa0, sh0, sw0, sa1, sh1, sw1, sa2, sh2, sw2):
        cid = lax.axis_index("c")
        sid = lax.axis_index("s")
        wid = sid * NC + cid

        pltpu.sync_copy(src_hbm.at[wid], src_v)
        pltpu.sync_copy(dst_hbm.at[wid], dst_v)
        pltpu.sync_copy(z16_hbm, den_sh.at[pl.ds(sid * STRIPE, STRIPE)])
        pltpu.sync_copy(zD_hbm, acc_sh.at[pl.ds(sid * STRIPE, STRIPE)])
        plsc.subcore_barrier()

        bufs = [(as0_v, ad0_v, h0_v, ex0_v, sa0, sh0, sw0),
                (as1_v, ad1_v, h1_v, ex1_v, sa1, sh1, sw1),
                (as2_v, ad2_v, h2_v, ex2_v, sa2, sh2, sw2)]
        pend_g = {}
        pend_w = {}

        def prefetch(j, b):
            as_v, ad_v, h_v, _, sem_a, sem_h, _ = bufs[b]
            pend_g[b] = (
                pltpu.async_copy(asrc_hbm.at[src_v.at[j]], as_v, sem_a),
                pltpu.async_copy(adst_hbm.at[dst_v.at[j]], ad_v, sem_a),
                pltpu.async_copy(h_hbm.at[src_v.at[j]], h_v, sem_h),
            )

        prefetch(0, 0)
        for j in range(NCHUNK):
            b = j % NB
            nb = (j + 1) % NB
            if j + 1 < NCHUNK:
                if nb in pend_w:
                    for w in pend_w.pop(nb):
                        w.wait()
                prefetch(j + 1, nb)
            as_v, ad_v, h_v, ex_v, _, _, sem_w = bufs[b]
            ca, cb, chh = pend_g[b]
            ca.wait()
            cb.wait()
            chh.wait()

            @plsc.parallel_loop(0, CH, unroll=2)
            def floop(i):
                s = as_v[i, :] + ad_v[i, :]
                c = jnp.exp(jnp.maximum(s, 0.2 * s))
                ex_v[i, :] = c
                for q in range(DV):
                    h_v[i, pl.ds(q * 16, 16)] = h_v[i, pl.ds(q * 16, 16)] * c

            pend_w[b] = (
                pltpu.async_copy(ex_v, den_sh.at[dst_v.at[j]], sem_w, add=True),
                pltpu.async_copy(h_v, acc_sh.at[dst_v.at[j]], sem_w, add=True),
            )

        for b in list(pend_w):
            for w in pend_w.pop(b):
                w.wait()
        plsc.subcore_barrier()
        pltpu.sync_copy(den_sh.at[pl.ds(sid * STRIPE, STRIPE)],
                        den_hbm.at[cid, pl.ds(sid * STRIPE, STRIPE)])
        pltpu.sync_copy(acc_sh.at[pl.ds(sid * STRIPE, STRIPE)],
                        acc_hbm.at[cid, pl.ds(sid * STRIPE, STRIPE)])

    vm = pltpu.VMEM
    return pl.kernel(
        body,
        out_type=[
            jax.ShapeDtypeStruct((NC, N_PAD, 16), jnp.float32),
            jax.ShapeDtypeStruct((NC, N_PAD, D), jnp.float32),
        ],
        mesh=_sc_mesh(),
        scratch_types=[
            vm((NCHUNK, CH), jnp.int32),
            vm((NCHUNK, CH), jnp.int32),
            vm((CH, 16), jnp.float32), vm((CH, 16), jnp.float32),
            vm((CH, D), jnp.float32), vm((CH, 16), jnp.float32),
            vm((CH, 16), jnp.float32), vm((CH, 16), jnp.float32),
            vm((CH, D), jnp.float32), vm((CH, 16), jnp.float32),
            vm((CH, 16), jnp.float32), vm((CH, 16), jnp.float32),
            vm((CH, D), jnp.float32), vm((CH, 16), jnp.float32),
            pltpu.VMEM_SHARED((N_PAD, 16), jnp.float32),
            pltpu.VMEM_SHARED((N_PAD, D), jnp.float32),
            pltpu.SemaphoreType.DMA, pltpu.SemaphoreType.DMA,
            pltpu.SemaphoreType.DMA, pltpu.SemaphoreType.DMA,
            pltpu.SemaphoreType.DMA, pltpu.SemaphoreType.DMA,
            pltpu.SemaphoreType.DMA, pltpu.SemaphoreType.DMA,
            pltpu.SemaphoreType.DMA,
        ],
        compiler_params=_sc_params,
    )


_edge_pass = functools.lru_cache(maxsize=None)(_make_edge_pass)


def _tc1_body(x_ref, w1_ref, asm_ref, adm_ref,
              h1_ref, as16_ref, ad16_ref, exs_ref):
    h1 = jnp.dot(x_ref[...], w1_ref[...], preferred_element_type=jnp.float32)
    h1_ref[...] = h1
    a_s = jnp.dot(h1, asm_ref[...], preferred_element_type=jnp.float32)
    a_d = jnp.dot(h1, adm_ref[...], preferred_element_type=jnp.float32)
    as16_ref[...] = a_s
    ad16_ref[...] = a_d
    s = a_s + a_d
    exs_ref[...] = jnp.exp(jnp.maximum(s, 0.2 * s))


def _tc1(x_pad, w1p, asm, adm):
    return pl.pallas_call(
        _tc1_body,
        grid=(NBLK,),
        in_specs=[
            pl.BlockSpec((ROWB, IN_CH), lambda i: (i, 0)),
            pl.BlockSpec((IN_CH, 64), lambda i: (0, 0)),
            pl.BlockSpec((64, 16), lambda i: (0, 0)),
            pl.BlockSpec((64, 16), lambda i: (0, 0)),
        ],
        out_specs=[
            pl.BlockSpec((ROWB, 64), lambda i: (i, 0)),
            pl.BlockSpec((ROWB, 16), lambda i: (i, 0)),
            pl.BlockSpec((ROWB, 16), lambda i: (i, 0)),
            pl.BlockSpec((ROWB, 16), lambda i: (i, 0)),
        ],
        out_shape=[
            jax.ShapeDtypeStruct((N_PAD, 64), jnp.float32),
            jax.ShapeDtypeStruct((N_PAD, 16), jnp.float32),
            jax.ShapeDtypeStruct((N_PAD, 16), jnp.float32),
            jax.ShapeDtypeStruct((N_PAD, 16), jnp.float32),
        ],
    )(x_pad, w1p, asm, adm)


def _tc2_body(denp_ref, accp_ref, h1_ref, exs_ref,
              b1_ref, w2_ref, a2sm_ref, a2dm_ref,
              h2_ref, a2s_ref, a2d_ref, exs2_ref):
    exs = exs_ref[...]
    den = denp_ref[0] + denp_ref[1] + exs
    num16 = h1_ref[...]
    coef = exs / (den + 1e-16)
    rden = 1.0 / (den + 1e-16)
    coef64 = jnp.concatenate([coef, coef, coef, coef], axis=1)
    rden64 = jnp.concatenate([rden, rden, rden, rden], axis=1)
    acc = accp_ref[0] + accp_ref[1]
    out1 = acc * rden64 + num16 * coef64 + b1_ref[...]
    hin2 = jnp.where(out1 > 0, out1, jnp.exp(jnp.minimum(out1, 0.0)) - 1.0)
    h2 = jnp.dot(hin2, w2_ref[...], preferred_element_type=jnp.float32)
    h2_ref[...] = h2
    a2s = jnp.dot(h2, a2sm_ref[...], preferred_element_type=jnp.float32)
    a2d = jnp.dot(h2, a2dm_ref[...], preferred_element_type=jnp.float32)
    a2s_ref[...] = a2s
    a2d_ref[...] = a2d
    s = a2s + a2d
    exs2_ref[...] = jnp.exp(jnp.maximum(s, 0.2 * s))


def _tc2(denp1, accp1, h1p, exs1, b1p, w2p, a2sm, a2dm):
    return pl.pallas_call(
        _tc2_body,
        grid=(NBLK,),
        in_specs=[
            pl.BlockSpec((NC, ROWB, 16), lambda i: (0, i, 0)),
            pl.BlockSpec((NC, ROWB, 64), lambda i: (0, i, 0)),
            pl.BlockSpec((ROWB, 64), lambda i: (i, 0)),
            pl.BlockSpec((ROWB, 16), lambda i: (i, 0)),
            pl.BlockSpec((1, 64), lambda i: (0, 0)),
            pl.BlockSpec((64, 16), lambda i: (0, 0)),
            pl.BlockSpec((16, 16), lambda i: (0, 0)),
            pl.BlockSpec((16, 16), lambda i: (0, 0)),
        ],
        out_specs=[
            pl.BlockSpec((ROWB, 16), lambda i: (i, 0)),
            pl.BlockSpec((ROWB, 16), lambda i: (i, 0)),
            pl.BlockSpec((ROWB, 16), lambda i: (i, 0)),
            pl.BlockSpec((ROWB, 16), lambda i: (i, 0)),
        ],
        out_shape=[
            jax.ShapeDtypeStruct((N_PAD, 16), jnp.float32),
            jax.ShapeDtypeStruct((N_PAD, 16), jnp.float32),
            jax.ShapeDtypeStruct((N_PAD, 16), jnp.float32),
            jax.ShapeDtypeStruct((N_PAD, 16), jnp.float32),
        ],
    )(denp1, accp1, h1p, exs1, b1p, w2p, a2sm, a2dm)


def _tc3_body(denp_ref, accp_ref, h2_ref, exs2_ref,
              b2_ref, out_ref):
    exs2 = exs2_ref[...]
    den = denp_ref[0] + denp_ref[1] + exs2
    rden = 1.0 / (den + 1e-16)
    acc = accp_ref[0] + accp_ref[1]
    out_ref[...] = acc * rden + h2_ref[...] * (exs2 * rden) + b2_ref[...]


def _tc3(denp2, accp2, h2, exs2, b2):
    return pl.pallas_call(
        _tc3_body,
        grid=(NBLK,),
        in_specs=[
            pl.BlockSpec((NC, ROWB, 16), lambda i: (0, i, 0)),
            pl.BlockSpec((NC, ROWB, 16), lambda i: (0, i, 0)),
            pl.BlockSpec((ROWB, 16), lambda i: (i, 0)),
            pl.BlockSpec((ROWB, 16), lambda i: (i, 0)),
            pl.BlockSpec((1, 16), lambda i: (0, 0)),
        ],
        out_specs=pl.BlockSpec((ROWB, 16), lambda i: (i, 0)),
        out_shape=jax.ShapeDtypeStruct((N_PAD, 16), jnp.float32),
    )(denp2, accp2, h2, exs2, b2)


_PERM = np.arange(64).reshape(HEADS, HID).T.reshape(-1)
_MASK16 = (np.arange(64)[:, None] % 8 == np.arange(16)[None, :] % 8)
_MASK16 = _MASK16.astype(np.float32)


def kernel(x, edge_index, W1, att_src1, att_dst1, b1, W2, att_src2, att_dst2, b2):
    f32 = jnp.float32
    x_pad = jnp.zeros((N_PAD, IN_CH), f32).at[:N].set(x)
    w1p = W1[:, _PERM]
    b1p = b1[_PERM].reshape(1, 64)
    w2p = W2[_PERM, :]
    att_s1p = att_src1[0].T.reshape(64)
    att_d1p = att_dst1[0].T.reshape(64)
    asm = att_s1p[:, None] * _MASK16
    adm = att_d1p[:, None] * _MASK16
    a2sm = jnp.broadcast_to(att_src2[0, 0][:, None], (16, 16)).astype(f32)
    a2dm = jnp.broadcast_to(att_dst2[0, 0][:, None], (16, 16)).astype(f32)

    src = edge_index[0].astype(jnp.int32)
    dst = edge_index[1].astype(jnp.int32)
    padv = jnp.full((E_PAD - E,), N, jnp.int32)
    srcJ = jnp.concatenate([src, padv]).reshape(NW, NCHUNK, CH)
    dstJ = jnp.concatenate([dst, padv]).reshape(NW, NCHUNK, CH)

    z16 = jnp.zeros((STRIPE, 16), f32)
    z64 = jnp.zeros((STRIPE, 64), f32)

    h1p, as16, ad16, exs1 = _tc1(x_pad, w1p, asm, adm)
    denp1, accp1 = _edge_pass(64)(as16, ad16, h1p, srcJ, dstJ, z16, z64)
    h2, a2s16, a2d16, exs2 = _tc2(denp1, accp1, h1p, exs1, b1p, w2p, a2sm, a2dm)

    denp2, accp2 = _edge_pass(16)(a2s16, a2d16, h2, srcJ, dstJ, z16, z16)
    out = _tc3(denp2, accp2, h2, exs2, b2.reshape(1, 16).astype(f32))
    return out[:N]
